# Initial kernel scaffold; baseline (speedup 1.0000x reference)
import jax
import jax.numpy as jnp
from jax.experimental import pallas as pl


def kernel(input_ids, table, A, B, gamma, beta):
    raise NotImplementedError("write your pallas kernel here")


# speedup vs baseline: 1.9260x; 1.9260x over previous
import functools

import jax
import jax.numpy as jnp
from jax import lax
from jax.experimental import pallas as pl
from jax.experimental.pallas import tpu as pltpu
from jax.experimental.pallas import tpu_sc as plsc

_CHUNK = 128


def _sc_gather(table, ids_flat):
    n = ids_flat.shape[0]
    d = table.shape[1]
    info = plsc.get_sparse_core_info()
    nw = info.num_cores * info.num_subcores
    assert n % (nw * _CHUNK) == 0
    per_w = n // nw
    n_chunks = per_w // _CHUNK

    mesh = plsc.VectorSubcoreMesh(core_axis_name="c", subcore_axis_name="s")

    @functools.partial(
        pl.kernel,
        mesh=mesh,
        out_type=jax.ShapeDtypeStruct((n, d), jnp.float32),
        scratch_types=[
            pltpu.VMEM((per_w,), jnp.int32),
            pltpu.VMEM((_CHUNK, d), jnp.float32),
            pltpu.SemaphoreType.DMA,
        ],
        compiler_params=pltpu.CompilerParams(use_tc_tiling_on_sc=False),
    )
    def k(table_hbm, idx_hbm, out_hbm, idx_v, buf, sem):
        wid = lax.axis_index("s") * info.num_cores + lax.axis_index("c")
        base = wid * per_w
        pltpu.sync_copy(idx_hbm.at[pl.ds(base, per_w)], idx_v)

        def body(j, carry):
            off = pl.multiple_of(j * _CHUNK, _CHUNK)
            idx_chunk = idx_v.at[pl.ds(off, _CHUNK)]
            pltpu.async_copy(table_hbm.at[idx_chunk], buf, sem).wait()
            out_off = pl.multiple_of(base + off, _CHUNK)
            pltpu.sync_copy(buf, out_hbm.at[pl.ds(out_off, _CHUNK)])
            return carry

        lax.fori_loop(0, n_chunks, body, 0, unroll=False)

    return k(table, ids_flat)


def _tc_transform(emb, At, Bt, gamma2d, beta2d):
    n, d_in = emb.shape
    d_mid = At.shape[1]
    d_out = Bt.shape[1]
    blk = 1024
    assert n % blk == 0

    def body(emb_ref, at_ref, bt_ref, g_ref, b_ref, out_ref):
        x = emb_ref[...]
        h = jnp.dot(x, at_ref[...], preferred_element_type=jnp.float32)
        h = jnp.dot(h, bt_ref[...], preferred_element_type=jnp.float32)
        mean = jnp.mean(h, axis=-1, keepdims=True)
        diff = h - mean
        var = jnp.mean(diff * diff, axis=-1, keepdims=True)
        y = diff * lax.rsqrt(var + 1e-5)
        out_ref[...] = y * g_ref[...] + b_ref[...]

    return pl.pallas_call(
        body,
        grid=(n // blk,),
        in_specs=[
            pl.BlockSpec((blk, d_in), lambda i: (i, 0)),
            pl.BlockSpec((d_in, d_mid), lambda i: (0, 0)),
            pl.BlockSpec((d_mid, d_out), lambda i: (0, 0)),
            pl.BlockSpec((1, d_out), lambda i: (0, 0)),
            pl.BlockSpec((1, d_out), lambda i: (0, 0)),
        ],
        out_specs=pl.BlockSpec((blk, d_out), lambda i: (i, 0)),
        out_shape=jax.ShapeDtypeStruct((n, d_out), jnp.float32),
    )(emb, At, Bt, gamma2d, beta2d)


def kernel(input_ids, table, A, B, gamma, beta):
    bsz, seq = input_ids.shape
    d_out = B.shape[0]
    ids = input_ids.reshape(-1).astype(jnp.int32)
    emb = _sc_gather(table, ids)
    out = _tc_transform(
        emb, A.T, B.T, gamma.reshape(1, d_out), beta.reshape(1, d_out)
    )
    return out.reshape(bsz, seq, d_out)


# speedup vs baseline: 2.6333x; 1.3672x over previous
import functools

import jax
import jax.numpy as jnp
from jax import lax
from jax.experimental import pallas as pl
from jax.experimental.pallas import tpu as pltpu
from jax.experimental.pallas import tpu_sc as plsc

_CHUNK = 128


def _tc_transform_table(table, At, Bt, gamma2d, beta2d):
    v, d_in = table.shape
    d_mid = At.shape[1]
    d_out = Bt.shape[1]
    blk = 1000
    assert v % blk == 0

    def body(x_ref, at_ref, bt_ref, g_ref, b_ref, out_ref):
        x = x_ref[...]
        h = jnp.dot(x, at_ref[...], preferred_element_type=jnp.float32)
        h = jnp.dot(h, bt_ref[...], preferred_element_type=jnp.float32)
        mean = jnp.mean(h, axis=-1, keepdims=True)
        diff = h - mean
        var = jnp.mean(diff * diff, axis=-1, keepdims=True)
        y = diff * lax.rsqrt(var + 1e-5)
        out_ref[...] = y * g_ref[...] + b_ref[...]

    return pl.pallas_call(
        body,
        grid=(v // blk,),
        in_specs=[
            pl.BlockSpec((blk, d_in), lambda i: (i, 0)),
            pl.BlockSpec((d_in, d_mid), lambda i: (0, 0)),
            pl.BlockSpec((d_mid, d_out), lambda i: (0, 0)),
            pl.BlockSpec((1, d_out), lambda i: (0, 0)),
            pl.BlockSpec((1, d_out), lambda i: (0, 0)),
        ],
        out_specs=pl.BlockSpec((blk, d_out), lambda i: (i, 0)),
        out_shape=jax.ShapeDtypeStruct((v, d_out), jnp.float32),
    )(table, At, Bt, gamma2d, beta2d)


def _sc_gather(table, ids_flat):
    n = ids_flat.shape[0]
    d = table.shape[1]
    info = plsc.get_sparse_core_info()
    nw = info.num_cores * info.num_subcores
    assert n % (nw * _CHUNK) == 0
    per_w = n // nw
    n_chunks = per_w // _CHUNK

    nbuf = 5
    assert n_chunks % nbuf == 0
    n_groups = n_chunks // nbuf

    mesh = plsc.VectorSubcoreMesh(core_axis_name="c", subcore_axis_name="s")

    @functools.partial(
        pl.kernel,
        mesh=mesh,
        out_type=jax.ShapeDtypeStruct((n, d), jnp.float32),
        scratch_types=[
            pltpu.VMEM((per_w,), jnp.int32),
            pltpu.VMEM((nbuf, _CHUNK, d), jnp.float32),
            pltpu.SemaphoreType.DMA((nbuf,)),
            pltpu.SemaphoreType.DMA((nbuf,)),
        ],
    )
    def k(table_hbm, idx_hbm, out_hbm, idx_v, bufs, gsems, ssems):
        wid = lax.axis_index("s") * info.num_cores + lax.axis_index("c")
        base = wid * per_w
        pltpu.sync_copy(idx_hbm.at[pl.ds(base, per_w)], idx_v)

        def out_slot(g, b):
            off = pl.multiple_of(base + (g * nbuf + b) * _CHUNK, _CHUNK)
            return out_hbm.at[pl.ds(off, _CHUNK)]

        def gather(g, b):
            off = pl.multiple_of((g * nbuf + b) * _CHUNK, _CHUNK)
            idx_chunk = idx_v.at[pl.ds(off, _CHUNK)]
            pltpu.async_copy(table_hbm.at[idx_chunk], bufs.at[b], gsems.at[b])

        @pl.loop(0, n_groups)
        def group(g):
            for b in range(nbuf):
                @pl.when(g > 0)
                def _():
                    pltpu.make_async_copy(
                        bufs.at[b], out_slot(g - 1, b), ssems.at[b]
                    ).wait()

                gather(g, b)
            for b in range(nbuf):
                pltpu.make_async_copy(
                    table_hbm.at[idx_v.at[pl.ds(0, _CHUNK)]], bufs.at[b], gsems.at[b]
                ).wait()
                pltpu.async_copy(bufs.at[b], out_slot(g, b), ssems.at[b])

        for b in range(nbuf):
            pltpu.make_async_copy(
                bufs.at[b], out_slot(n_groups - 1, b), ssems.at[b]
            ).wait()

    return k(table, ids_flat)


def kernel(input_ids, table, A, B, gamma, beta):
    bsz, seq = input_ids.shape
    d_out = B.shape[0]
    transformed = _tc_transform_table(
        table, A.T, B.T, gamma.reshape(1, d_out), beta.reshape(1, d_out)
    )
    ids = input_ids.reshape(-1).astype(jnp.int32)
    out = _sc_gather(transformed, ids)
    return out.reshape(bsz, seq, d_out)


# speedup vs baseline: 5.9592x; 2.2630x over previous
import functools

import jax
import jax.numpy as jnp
from jax import lax
from jax.experimental import pallas as pl
from jax.experimental.pallas import tpu as pltpu
from jax.experimental.pallas import tpu_sc as plsc

_CHUNK = 128


def _tc_transform_table(tableT, A, B, gamma2d, beta2d):
    d_in, v = tableT.shape
    d_mid, _ = A.shape
    d_out, _ = B.shape
    blk = 1024
    grid = (v + blk - 1) // blk

    def body(xt_ref, a_ref, b_ref, g_ref, be_ref, out_ref):
        c = jnp.dot(a_ref[...].T, b_ref[...].T, preferred_element_type=jnp.float32)
        xt = xt_ref[...]
        h = lax.dot_general(
            xt, c, (((0,), (0,)), ((), ())), preferred_element_type=jnp.float32
        )
        mean = jnp.mean(h, axis=-1, keepdims=True)
        diff = h - mean
        var = jnp.mean(diff * diff, axis=-1, keepdims=True)
        y = diff * lax.rsqrt(var + 1e-5)
        out_ref[...] = y * g_ref[...] + be_ref[...]

    return pl.pallas_call(
        body,
        grid=(grid,),
        in_specs=[
            pl.BlockSpec((d_in, blk), lambda i: (0, i)),
            pl.BlockSpec((d_mid, d_in), lambda i: (0, 0)),
            pl.BlockSpec((d_out, d_mid), lambda i: (0, 0)),
            pl.BlockSpec((1, d_out), lambda i: (0, 0)),
            pl.BlockSpec((1, d_out), lambda i: (0, 0)),
        ],
        out_specs=pl.BlockSpec((blk, d_out), lambda i: (i, 0)),
        out_shape=jax.ShapeDtypeStruct((v, d_out), jnp.float32),
    )(tableT, A, B, gamma2d, beta2d)


def _sc_gather(table, ids_flat):
    n = ids_flat.shape[0]
    d = table.shape[1]
    info = plsc.get_sparse_core_info()
    nw = info.num_cores * info.num_subcores
    assert n % (nw * _CHUNK) == 0
    per_w = n // nw
    n_chunks = per_w // _CHUNK

    nbuf = 5
    assert n_chunks % nbuf == 0
    n_groups = n_chunks // nbuf

    mesh = plsc.VectorSubcoreMesh(core_axis_name="c", subcore_axis_name="s")

    @functools.partial(
        pl.kernel,
        mesh=mesh,
        out_type=jax.ShapeDtypeStruct((n, d), jnp.float32),
        scratch_types=[
            pltpu.VMEM((per_w,), jnp.int32),
            pltpu.VMEM((nbuf, _CHUNK, d), jnp.float32),
            pltpu.SemaphoreType.DMA((nbuf,)),
            pltpu.SemaphoreType.DMA((nbuf,)),
        ],
        compiler_params=pltpu.CompilerParams(use_tc_tiling_on_sc=True),
    )
    def k(table_hbm, idx_hbm, out_hbm, idx_v, bufs, gsems, ssems):
        wid = lax.axis_index("s") * info.num_cores + lax.axis_index("c")
        base = wid * per_w
        pltpu.sync_copy(idx_hbm.at[pl.ds(base, per_w)], idx_v)

        def out_slot(g, b):
            off = pl.multiple_of(base + (g * nbuf + b) * _CHUNK, _CHUNK)
            return out_hbm.at[pl.ds(off, _CHUNK)]

        def gather(g, b):
            off = pl.multiple_of((g * nbuf + b) * _CHUNK, _CHUNK)
            idx_chunk = idx_v.at[pl.ds(off, _CHUNK)]
            pltpu.async_copy(table_hbm.at[idx_chunk], bufs.at[b], gsems.at[b])

        @pl.loop(0, n_groups)
        def group(g):
            for b in range(nbuf):
                @pl.when(g > 0)
                def _():
                    pltpu.make_async_copy(
                        bufs.at[b], out_slot(g - 1, b), ssems.at[b]
                    ).wait()

                gather(g, b)
            for b in range(nbuf):
                pltpu.make_async_copy(
                    table_hbm.at[idx_v.at[pl.ds(0, _CHUNK)]], bufs.at[b], gsems.at[b]
                ).wait()
                pltpu.async_copy(bufs.at[b], out_slot(g, b), ssems.at[b])

        for b in range(nbuf):
            pltpu.make_async_copy(
                bufs.at[b], out_slot(n_groups - 1, b), ssems.at[b]
            ).wait()

    return k(table, ids_flat)


def kernel(input_ids, table, A, B, gamma, beta):
    bsz, seq = input_ids.shape
    d_out = B.shape[0]
    transformed = _tc_transform_table(
        table.T, A, B, gamma.reshape(1, d_out), beta.reshape(1, d_out)
    )
    ids = input_ids.T.reshape(-1).astype(jnp.int32)
    out = _sc_gather(transformed, ids)
    return out.reshape(seq, bsz, d_out).transpose(1, 0, 2)


# speedup vs baseline: 7.6399x; 1.2820x over previous
import functools

import jax
import jax.numpy as jnp
from jax import lax
from jax.experimental import pallas as pl
from jax.experimental.pallas import tpu as pltpu
from jax.experimental.pallas import tpu_sc as plsc

_CHUNK = 128


def _tc_transform_table(tableT, A, B, gamma2d, beta2d):
    d_in, v = tableT.shape
    d_mid, _ = A.shape
    d_out, _ = B.shape
    blk = 4096
    grid = (v + blk - 1) // blk

    def body(xt_ref, a_ref, b_ref, g_ref, be_ref, out_ref):
        c = jnp.dot(a_ref[...].T, b_ref[...].T, preferred_element_type=jnp.float32)
        xt = xt_ref[...]
        h = lax.dot_general(
            xt, c, (((0,), (0,)), ((), ())), preferred_element_type=jnp.float32
        )
        mean = jnp.mean(h, axis=-1, keepdims=True)
        diff = h - mean
        var = jnp.mean(diff * diff, axis=-1, keepdims=True)
        y = diff * lax.rsqrt(var + 1e-5)
        out_ref[...] = y * g_ref[...] + be_ref[...]

    return pl.pallas_call(
        body,
        grid=(grid,),
        in_specs=[
            pl.BlockSpec((d_in, blk), lambda i: (0, i)),
            pl.BlockSpec((d_mid, d_in), lambda i: (0, 0)),
            pl.BlockSpec((d_out, d_mid), lambda i: (0, 0)),
            pl.BlockSpec((1, d_out), lambda i: (0, 0)),
            pl.BlockSpec((1, d_out), lambda i: (0, 0)),
        ],
        out_specs=pl.BlockSpec((blk, d_out), lambda i: (i, 0)),
        out_shape=jax.ShapeDtypeStruct((v, d_out), jnp.float32),
    )(tableT, A, B, gamma2d, beta2d)


def _sc_gather(table, ids_flat):
    n = ids_flat.shape[0]
    d = table.shape[1]
    info = plsc.get_sparse_core_info()
    nw = info.num_cores * info.num_subcores
    assert n % (nw * _CHUNK) == 0
    per_w = n // nw
    n_chunks = per_w // _CHUNK

    nbuf = 5
    assert n_chunks % nbuf == 0
    n_groups = n_chunks // nbuf

    mesh = plsc.VectorSubcoreMesh(core_axis_name="c", subcore_axis_name="s")

    @functools.partial(
        pl.kernel,
        mesh=mesh,
        out_type=jax.ShapeDtypeStruct((n, d), jnp.float32),
        scratch_types=[
            pltpu.VMEM((per_w,), jnp.int32),
            pltpu.VMEM((nbuf, _CHUNK, d), jnp.float32),
            pltpu.SemaphoreType.DMA((nbuf,)),
            pltpu.SemaphoreType.DMA((nbuf,)),
        ],
        compiler_params=pltpu.CompilerParams(use_tc_tiling_on_sc=True),
    )
    def k(table_hbm, idx_hbm, out_hbm, idx_v, bufs, gsems, ssems):
        wid = lax.axis_index("s") * info.num_cores + lax.axis_index("c")
        base = wid * per_w
        pltpu.sync_copy(idx_hbm.at[pl.ds(base, per_w)], idx_v)

        def out_slot(g, b):
            off = pl.multiple_of(base + (g * nbuf + b) * _CHUNK, _CHUNK)
            return out_hbm.at[pl.ds(off, _CHUNK)]

        def gather(g, b):
            off = pl.multiple_of((g * nbuf + b) * _CHUNK, _CHUNK)
            idx_chunk = idx_v.at[pl.ds(off, _CHUNK)]
            pltpu.async_copy(table_hbm.at[idx_chunk], bufs.at[b], gsems.at[b])

        @pl.loop(0, n_groups)
        def group(g):
            for b in range(nbuf):
                @pl.when(g > 0)
                def _():
                    pltpu.make_async_copy(
                        bufs.at[b], out_slot(g - 1, b), ssems.at[b]
                    ).wait()

                gather(g, b)
            for b in range(nbuf):
                pltpu.make_async_copy(
                    table_hbm.at[idx_v.at[pl.ds(0, _CHUNK)]], bufs.at[b], gsems.at[b]
                ).wait()
                pltpu.async_copy(bufs.at[b], out_slot(g, b), ssems.at[b])

        for b in range(nbuf):
            pltpu.make_async_copy(
                bufs.at[b], out_slot(n_groups - 1, b), ssems.at[b]
            ).wait()

    return k(table, ids_flat)


def kernel(input_ids, table, A, B, gamma, beta):
    bsz, seq = input_ids.shape
    d_out = B.shape[0]
    transformed = _tc_transform_table(
        table.T, A, B, gamma.reshape(1, d_out), beta.reshape(1, d_out)
    )
    ids = input_ids.T.reshape(-1).astype(jnp.int32)
    out = _sc_gather(transformed, ids)
    return out.reshape(seq, bsz, d_out).transpose(1, 0, 2)


# speedup vs baseline: 7.8372x; 1.0258x over previous
import functools

import jax
import jax.numpy as jnp
from jax import lax
from jax.experimental import pallas as pl
from jax.experimental.pallas import tpu as pltpu
from jax.experimental.pallas import tpu_sc as plsc

_CHUNK = 128


def _tc_transform_table(tableT, At, Bt, gamma2d, beta2d):
    d_in, v = tableT.shape
    _, d_mid = At.shape
    _, d_out = Bt.shape
    blk = 8192
    grid = (v + blk - 1) // blk

    def body(xt_ref, a_ref, b_ref, g_ref, be_ref, out_ref):
        c = jnp.dot(a_ref[...], b_ref[...], preferred_element_type=jnp.float32)
        xt = xt_ref[...]
        h = lax.dot_general(
            xt, c, (((0,), (0,)), ((), ())), preferred_element_type=jnp.float32
        )
        mean = jnp.mean(h, axis=-1, keepdims=True)
        diff = h - mean
        var = jnp.mean(diff * diff, axis=-1, keepdims=True)
        y = diff * lax.rsqrt(var + 1e-5)
        out_ref[...] = y * g_ref[...] + be_ref[...]

    return pl.pallas_call(
        body,
        grid=(grid,),
        in_specs=[
            pl.BlockSpec((d_in, blk), lambda i: (0, i)),
            pl.BlockSpec((d_in, d_mid), lambda i: (0, 0)),
            pl.BlockSpec((d_mid, d_out), lambda i: (0, 0)),
            pl.BlockSpec((1, d_out), lambda i: (0, 0)),
            pl.BlockSpec((1, d_out), lambda i: (0, 0)),
        ],
        out_specs=pl.BlockSpec((blk, d_out), lambda i: (i, 0)),
        out_shape=jax.ShapeDtypeStruct((v, d_out), jnp.float32),
    )(tableT, At, Bt, gamma2d, beta2d)


def _sc_gather(table, ids_flat):
    n = ids_flat.shape[0]
    d = table.shape[1]
    info = plsc.get_sparse_core_info()
    nw = info.num_cores * info.num_subcores
    assert n % (nw * _CHUNK) == 0
    per_w = n // nw
    n_chunks = per_w // _CHUNK

    nbuf = 5
    assert n_chunks % nbuf == 0
    n_groups = n_chunks // nbuf

    mesh = plsc.VectorSubcoreMesh(core_axis_name="c", subcore_axis_name="s")

    @functools.partial(
        pl.kernel,
        mesh=mesh,
        out_type=jax.ShapeDtypeStruct((n, d), jnp.float32),
        scratch_types=[
            pltpu.VMEM((per_w,), jnp.int32),
            pltpu.VMEM((nbuf, _CHUNK, d), jnp.float32),
            pltpu.SemaphoreType.DMA((nbuf,)),
            pltpu.SemaphoreType.DMA((nbuf,)),
        ],
        compiler_params=pltpu.CompilerParams(use_tc_tiling_on_sc=True),
    )
    def k(table_hbm, idx_hbm, out_hbm, idx_v, bufs, gsems, ssems):
        wid = lax.axis_index("s") * info.num_cores + lax.axis_index("c")
        base = wid * per_w
        pltpu.sync_copy(idx_hbm.at[pl.ds(base, per_w)], idx_v)

        def out_slot(g, b):
            off = pl.multiple_of(base + (g * nbuf + b) * _CHUNK, _CHUNK)
            return out_hbm.at[pl.ds(off, _CHUNK)]

        def gather(g, b):
            off = pl.multiple_of((g * nbuf + b) * _CHUNK, _CHUNK)
            idx_chunk = idx_v.at[pl.ds(off, _CHUNK)]
            pltpu.async_copy(table_hbm.at[idx_chunk], bufs.at[b], gsems.at[b])

        @pl.loop(0, n_groups)
        def group(g):
            for b in range(nbuf):
                @pl.when(g > 0)
                def _():
                    pltpu.make_async_copy(
                        bufs.at[b], out_slot(g - 1, b), ssems.at[b]
                    ).wait()

                gather(g, b)
            for b in range(nbuf):
                pltpu.make_async_copy(
                    table_hbm.at[idx_v.at[pl.ds(0, _CHUNK)]], bufs.at[b], gsems.at[b]
                ).wait()
                pltpu.async_copy(bufs.at[b], out_slot(g, b), ssems.at[b])

        for b in range(nbuf):
            pltpu.make_async_copy(
                bufs.at[b], out_slot(n_groups - 1, b), ssems.at[b]
            ).wait()

    return k(table, ids_flat)


def kernel(input_ids, table, A, B, gamma, beta):
    bsz, seq = input_ids.shape
    d_out = B.shape[0]
    transformed = _tc_transform_table(
        table.T, A.T, B.T, gamma.reshape(1, d_out), beta.reshape(1, d_out)
    )
    ids = input_ids.T.reshape(-1).astype(jnp.int32)
    out = _sc_gather(transformed, ids)
    return out.reshape(seq, bsz, d_out).transpose(1, 0, 2)


# speedup vs baseline: 7.9564x; 1.0152x over previous
import functools

import jax
import jax.numpy as jnp
from jax import lax
from jax.experimental import pallas as pl
from jax.experimental.pallas import tpu as pltpu
from jax.experimental.pallas import tpu_sc as plsc

_CHUNK = 64


def _tc_transform_table(tableT, At, Bt, gamma2d, beta2d):
    d_in, v = tableT.shape
    _, d_mid = At.shape
    _, d_out = Bt.shape
    blk = 8192
    grid = (v + blk - 1) // blk

    def body(xt_ref, a_ref, b_ref, g_ref, be_ref, out_ref):
        c = jnp.dot(a_ref[...], b_ref[...], preferred_element_type=jnp.float32)
        xt = xt_ref[...]
        h = lax.dot_general(
            xt, c, (((0,), (0,)), ((), ())), preferred_element_type=jnp.float32
        )
        mean = jnp.mean(h, axis=-1, keepdims=True)
        diff = h - mean
        var = jnp.mean(diff * diff, axis=-1, keepdims=True)
        y = diff * lax.rsqrt(var + 1e-5)
        out_ref[...] = y * g_ref[...] + be_ref[...]

    return pl.pallas_call(
        body,
        grid=(grid,),
        in_specs=[
            pl.BlockSpec((d_in, blk), lambda i: (0, i)),
            pl.BlockSpec((d_in, d_mid), lambda i: (0, 0)),
            pl.BlockSpec((d_mid, d_out), lambda i: (0, 0)),
            pl.BlockSpec((1, d_out), lambda i: (0, 0)),
            pl.BlockSpec((1, d_out), lambda i: (0, 0)),
        ],
        out_specs=pl.BlockSpec((blk, d_out), lambda i: (i, 0)),
        out_shape=jax.ShapeDtypeStruct((v, d_out), jnp.float32),
    )(tableT, At, Bt, gamma2d, beta2d)


def _sc_gather(table, ids_flat):
    n = ids_flat.shape[0]
    d = table.shape[1]
    info = plsc.get_sparse_core_info()
    nw = info.num_cores * info.num_subcores
    assert n % (nw * _CHUNK) == 0
    per_w = n // nw
    n_chunks = per_w // _CHUNK

    nbuf = 10
    assert n_chunks % nbuf == 0
    n_groups = n_chunks // nbuf

    mesh = plsc.VectorSubcoreMesh(core_axis_name="c", subcore_axis_name="s")

    @functools.partial(
        pl.kernel,
        mesh=mesh,
        out_type=jax.ShapeDtypeStruct((n, d), jnp.float32),
        scratch_types=[
            pltpu.VMEM((per_w,), jnp.int32),
            pltpu.VMEM((nbuf, _CHUNK, d), jnp.float32),
            pltpu.SemaphoreType.DMA((nbuf,)),
            pltpu.SemaphoreType.DMA((nbuf,)),
        ],
        compiler_params=pltpu.CompilerParams(use_tc_tiling_on_sc=True),
    )
    def k(table_hbm, idx_hbm, out_hbm, idx_v, bufs, gsems, ssems):
        wid = lax.axis_index("s") * info.num_cores + lax.axis_index("c")
        base = wid * per_w
        pltpu.sync_copy(idx_hbm.at[pl.ds(base, per_w)], idx_v)

        def out_slot(g, b):
            off = pl.multiple_of(base + (g * nbuf + b) * _CHUNK, _CHUNK)
            return out_hbm.at[pl.ds(off, _CHUNK)]

        def gather(g, b):
            off = pl.multiple_of((g * nbuf + b) * _CHUNK, _CHUNK)
            idx_chunk = idx_v.at[pl.ds(off, _CHUNK)]
            pltpu.async_copy(table_hbm.at[idx_chunk], bufs.at[b], gsems.at[b])

        @pl.loop(0, n_groups)
        def group(g):
            for b in range(nbuf):
                @pl.when(g > 0)
                def _():
                    pltpu.make_async_copy(
                        bufs.at[b], out_slot(g - 1, b), ssems.at[b]
                    ).wait()

                gather(g, b)
            for b in range(nbuf):
                pltpu.make_async_copy(
                    table_hbm.at[idx_v.at[pl.ds(0, _CHUNK)]], bufs.at[b], gsems.at[b]
                ).wait()
                pltpu.async_copy(bufs.at[b], out_slot(g, b), ssems.at[b])

        for b in range(nbuf):
            pltpu.make_async_copy(
                bufs.at[b], out_slot(n_groups - 1, b), ssems.at[b]
            ).wait()

    return k(table, ids_flat)


def kernel(input_ids, table, A, B, gamma, beta):
    bsz, seq = input_ids.shape
    d_out = B.shape[0]
    transformed = _tc_transform_table(
        table.T, A.T, B.T, gamma.reshape(1, d_out), beta.reshape(1, d_out)
    )
    ids = input_ids.T.reshape(-1).astype(jnp.int32)
    out = _sc_gather(transformed, ids)
    return out.reshape(seq, bsz, d_out).transpose(1, 0, 2)


# speedup vs baseline: 8.6381x; 1.0857x over previous
import functools

import jax
import jax.numpy as jnp
from jax import lax
from jax.experimental import pallas as pl
from jax.experimental.pallas import tpu as pltpu
from jax.experimental.pallas import tpu_sc as plsc

_CHUNK = 64


def _tc_transform_table(tableT, At, Bt, gamma2d, beta2d):
    d_in, v = tableT.shape
    _, d_mid = At.shape
    _, d_out = Bt.shape
    blk = 8192
    grid = (v + blk - 1) // blk

    def body(xt_ref, a_ref, b_ref, g_ref, be_ref, out_ref):
        c = jnp.dot(a_ref[...], b_ref[...], preferred_element_type=jnp.float32)
        cm = jnp.mean(c, axis=1, keepdims=True)
        cc = c - cm
        xt = xt_ref[...]
        diff = lax.dot_general(
            xt, cc, (((0,), (0,)), ((), ())), preferred_element_type=jnp.float32
        )
        var = jnp.mean(diff * diff, axis=-1, keepdims=True)
        y = diff * lax.rsqrt(var + 1e-5)
        out_ref[...] = y * g_ref[...] + be_ref[...]

    return pl.pallas_call(
        body,
        grid=(grid,),
        in_specs=[
            pl.BlockSpec((d_in, blk), lambda i: (0, i)),
            pl.BlockSpec((d_in, d_mid), lambda i: (0, 0)),
            pl.BlockSpec((d_mid, d_out), lambda i: (0, 0)),
            pl.BlockSpec((1, d_out), lambda i: (0, 0)),
            pl.BlockSpec((1, d_out), lambda i: (0, 0)),
        ],
        out_specs=pl.BlockSpec((blk, d_out), lambda i: (i, 0)),
        out_shape=jax.ShapeDtypeStruct((v, d_out), jnp.float32),
    )(tableT, At, Bt, gamma2d, beta2d)


def _sc_gather(table, ids_flat):
    n = ids_flat.shape[0]
    d = table.shape[1]
    info = plsc.get_sparse_core_info()
    nw = info.num_cores * info.num_subcores
    assert n % (nw * _CHUNK) == 0
    per_w = n // nw
    n_chunks = per_w // _CHUNK

    nbuf = 10
    assert n_chunks % nbuf == 0
    n_groups = n_chunks // nbuf

    mesh = plsc.VectorSubcoreMesh(core_axis_name="c", subcore_axis_name="s")

    @functools.partial(
        pl.kernel,
        mesh=mesh,
        out_type=jax.ShapeDtypeStruct((n, d), jnp.float32),
        scratch_types=[
            pltpu.VMEM((per_w,), jnp.int32),
            pltpu.VMEM((nbuf, _CHUNK, d), jnp.float32),
            pltpu.SemaphoreType.DMA((nbuf,)),
            pltpu.SemaphoreType.DMA((nbuf,)),
        ],
        compiler_params=pltpu.CompilerParams(use_tc_tiling_on_sc=True),
    )
    def k(table_hbm, idx_hbm, out_hbm, idx_v, bufs, gsems, ssems):
        wid = lax.axis_index("s") * info.num_cores + lax.axis_index("c")
        base = wid * per_w
        pltpu.sync_copy(idx_hbm.at[pl.ds(base, per_w)], idx_v)

        def out_slot(g, b):
            off = pl.multiple_of(base + (g * nbuf + b) * _CHUNK, _CHUNK)
            return out_hbm.at[pl.ds(off, _CHUNK)]

        def gather(g, b):
            off = pl.multiple_of((g * nbuf + b) * _CHUNK, _CHUNK)
            idx_chunk = idx_v.at[pl.ds(off, _CHUNK)]
            pltpu.async_copy(table_hbm.at[idx_chunk], bufs.at[b], gsems.at[b])

        @pl.loop(0, n_groups)
        def group(g):
            for b in range(nbuf):
                @pl.when(g > 0)
                def _():
                    pltpu.make_async_copy(
                        bufs.at[b], out_slot(g - 1, b), ssems.at[b]
                    ).wait()

                gather(g, b)
            for b in range(nbuf):
                pltpu.make_async_copy(
                    table_hbm.at[idx_v.at[pl.ds(0, _CHUNK)]], bufs.at[b], gsems.at[b]
                ).wait()
                pltpu.async_copy(bufs.at[b], out_slot(g, b), ssems.at[b])

        for b in range(nbuf):
            pltpu.make_async_copy(
                bufs.at[b], out_slot(n_groups - 1, b), ssems.at[b]
            ).wait()

    return k(table, ids_flat)


def kernel(input_ids, table, A, B, gamma, beta):
    bsz, seq = input_ids.shape
    d_out = B.shape[0]
    transformed = _tc_transform_table(
        table.T, A.T, B.T, gamma.reshape(1, d_out), beta.reshape(1, d_out)
    )
    ids = input_ids.T.reshape(-1).astype(jnp.int32)
    out = _sc_gather(transformed, ids)
    return out.reshape(seq, bsz, d_out).transpose(1, 0, 2)


# speedup vs baseline: 8.8231x; 1.0214x over previous
import functools

import jax
import jax.numpy as jnp
from jax import lax
from jax.experimental import pallas as pl
from jax.experimental.pallas import tpu as pltpu
from jax.experimental.pallas import tpu_sc as plsc

_CHUNK = 128


def _tc_transform_table(tableT, At, Bt, gamma2d, beta2d):
    d_in, v = tableT.shape
    _, d_mid = At.shape
    _, d_out = Bt.shape
    blk = 8192
    grid = (v + blk - 1) // blk

    def body(xt_ref, a_ref, b_ref, g_ref, be_ref, out_ref):
        c = jnp.dot(a_ref[...], b_ref[...], preferred_element_type=jnp.float32)
        cm = jnp.mean(c, axis=1, keepdims=True)
        cc = c - cm
        xt = xt_ref[...]
        diff = lax.dot_general(
            xt, cc, (((0,), (0,)), ((), ())), preferred_element_type=jnp.float32
        )
        var = jnp.mean(diff * diff, axis=-1, keepdims=True)
        y = diff * lax.rsqrt(var + 1e-5)
        out_ref[...] = y * g_ref[...] + be_ref[...]

    return pl.pallas_call(
        body,
        grid=(grid,),
        in_specs=[
            pl.BlockSpec((d_in, blk), lambda i: (0, i)),
            pl.BlockSpec((d_in, d_mid), lambda i: (0, 0)),
            pl.BlockSpec((d_mid, d_out), lambda i: (0, 0)),
            pl.BlockSpec((1, d_out), lambda i: (0, 0)),
            pl.BlockSpec((1, d_out), lambda i: (0, 0)),
        ],
        out_specs=pl.BlockSpec((blk, d_out), lambda i: (i, 0)),
        out_shape=jax.ShapeDtypeStruct((v, d_out), jnp.float32),
    )(tableT, At, Bt, gamma2d, beta2d)


def _sc_gather(table, ids2d):
    s, bsz = ids2d.shape
    n = s * bsz
    d = table.shape[1]
    info = plsc.get_sparse_core_info()
    nw = info.num_cores * info.num_subcores
    assert bsz % (nw * _CHUNK) == 0
    n_chunks = s

    nbuf = 5
    assert n_chunks % nbuf == 0
    n_groups = n_chunks // nbuf

    mesh = plsc.VectorSubcoreMesh(core_axis_name="c", subcore_axis_name="s")

    @functools.partial(
        pl.kernel,
        mesh=mesh,
        out_type=jax.ShapeDtypeStruct((n, d), jnp.float32),
        scratch_types=[
            pltpu.VMEM((s, _CHUNK), jnp.int32),
            pltpu.VMEM((nbuf, _CHUNK, d), jnp.float32),
            pltpu.SemaphoreType.DMA((nbuf,)),
            pltpu.SemaphoreType.DMA((nbuf,)),
        ],
        compiler_params=pltpu.CompilerParams(use_tc_tiling_on_sc=True),
    )
    def k(table_hbm, idx_hbm, out_hbm, idx_v, bufs, gsems, ssems):
        wid = lax.axis_index("s") * info.num_cores + lax.axis_index("c")
        col = wid * _CHUNK
        pltpu.sync_copy(idx_hbm.at[pl.ds(0, s), pl.ds(col, _CHUNK)], idx_v)

        def out_slot(g, b):
            off = pl.multiple_of((g * nbuf + b) * bsz + col, _CHUNK)
            return out_hbm.at[pl.ds(off, _CHUNK)]

        def gather(g, b):
            idx_chunk = idx_v.at[g * nbuf + b]
            pltpu.async_copy(table_hbm.at[idx_chunk], bufs.at[b], gsems.at[b])

        @pl.loop(0, n_groups)
        def group(g):
            for b in range(nbuf):
                @pl.when(g > 0)
                def _():
                    pltpu.make_async_copy(
                        bufs.at[b], out_slot(g - 1, b), ssems.at[b]
                    ).wait()

                gather(g, b)
            for b in range(nbuf):
                pltpu.make_async_copy(
                    table_hbm.at[idx_v.at[0]], bufs.at[b], gsems.at[b]
                ).wait()
                pltpu.async_copy(bufs.at[b], out_slot(g, b), ssems.at[b])

        for b in range(nbuf):
            pltpu.make_async_copy(
                bufs.at[b], out_slot(n_groups - 1, b), ssems.at[b]
            ).wait()

    return k(table, ids2d)


def kernel(input_ids, table, A, B, gamma, beta):
    bsz, seq = input_ids.shape
    d_out = B.shape[0]
    transformed = _tc_transform_table(
        table.T, A.T, B.T, gamma.reshape(1, d_out), beta.reshape(1, d_out)
    )
    ids2d = input_ids.T.astype(jnp.int32)
    out = _sc_gather(transformed, ids2d)
    return out.reshape(seq, bsz, d_out).transpose(1, 0, 2)
